# baseline (speedup 1.0000x reference)
import functools

import jax
import jax.numpy as jnp
from jax import lax
from jax.experimental import pallas as pl
from jax.experimental.pallas import tpu as pltpu
from jax.experimental.pallas import tpu_sc as plsc

N = 50000
E = 800000
IN_DIM = 16
HID = 64
OUT = 3

NC = 2
NS = 16
NW = NC * NS
EPT = 25600
E_PAD = NW * EPT
L1_ROWS = 2 * N + 96
L2_ROWS = N + 48
ZROWS = 1564


def _edge_pass(acc_rows, gather_by_ik, ch, rw):
    rpt = acc_rows // NS
    nchunk = EPT // ch
    nvreg = ch // 16
    mesh = plsc.VectorSubcoreMesh(core_axis_name="c", subcore_axis_name="s")

    nbuf = 2
    buf_types = []
    for _ in range(nbuf):
        buf_types += [
            pltpu.VMEM((3, ch), jnp.int32),
            pltpu.VMEM((2 * ch,), jnp.int32),
            pltpu.VMEM((ch,), jnp.int32),
            pltpu.VMEM((ch,), jnp.int32),
            pltpu.VMEM((2 * ch,), jnp.float32),
            pltpu.VMEM((ch,), jnp.float32),
            pltpu.VMEM((ch, 16), jnp.float32)
            if rw == 16 else pltpu.VMEM((ch * rw,), jnp.float32),
            pltpu.VMEM((ch * rw,), jnp.int32)
            if rw != 16 else pltpu.VMEM((8,), jnp.int32),
            pltpu.VMEM((ch * rw,), jnp.int32)
            if rw != 16 else pltpu.VMEM((8,), jnp.int32),
            pltpu.SemaphoreType.DMA,
            pltpu.SemaphoreType.DMA,
            pltpu.SemaphoreType.DMA,
            pltpu.SemaphoreType.DMA,
        ]

    @functools.partial(
        pl.kernel,
        mesh=mesh,
        compiler_params=pltpu.CompilerParams(use_tc_tiling_on_sc=False),
        out_type=[
            jax.ShapeDtypeStruct((NC, acc_rows, 16), jnp.float32)
            if rw == 16 else
            jax.ShapeDtypeStruct((NC * acc_rows * rw,), jnp.float32),
            jax.ShapeDtypeStruct((NC * acc_rows,), jnp.float32),
        ],
        scratch_types=[
            pltpu.VMEM_SHARED((acc_rows, 16), jnp.float32)
            if rw == 16 else
            pltpu.VMEM_SHARED((acc_rows * rw,), jnp.float32),
            pltpu.VMEM_SHARED((acc_rows,), jnp.float32),
        ] + buf_types,
    )
    def kern(qkt, tab, e3, z2d, zfl, agg_out, den_out, agg_s, den_s, *bufs):
        names = ("ebuf", "ckb", "gidx", "sidx", "qkv", "ab", "rows",
                 "egidx", "eidx", "semE", "semQ", "semR", "semS")
        A = dict(zip(names, bufs[:len(names)]))
        B = dict(zip(names, bufs[len(names):]))
        c = lax.axis_index("c")
        s = lax.axis_index("s")
        wid = c * NS + s
        tbase = wid * EPT
        rbase = s * rpt
        gmax = 2 * N - 1

        def fire_idx(i, st):
            pltpu.async_copy(e3.at[:, pl.ds(tbase + i * ch, ch)],
                             st["ebuf"], st["semE"])

        def wait_idx(st):
            pltpu.make_async_copy(e3.at[:, pl.ds(tbase, ch)],
                                  st["ebuf"], st["semE"]).wait()

        def vix(st):
            ebuf, ckb, gidx, sidx = st["ebuf"], st["ckb"], st["gidx"], st["sidx"]
            def body(jv, cc):
                sl = pl.ds(jv * 16, 16)
                sr = ebuf[0, sl]
                d = ebuf[1, sl]
                t = ebuf[2, sl]
                iq = 2 * d + t
                ik = 2 * sr + t
                ckb[sl] = jnp.minimum(iq, gmax)
                ckb[pl.ds(ch + jv * 16, 16)] = 2 * N + ik
                if gather_by_ik:
                    gidx[sl] = ik
                    sidx[sl] = d
                else:
                    gidx[sl] = sr
                    sidx[sl] = iq
                if rw != 16:
                    eidx = st["eidx"]
                    egidx = st["egidx"]
                    iotav = lax.iota(jnp.int32, 16)
                    sub = iotav >> 2
                    lane = iotav & 3
                    for u in range(4):
                        sl4 = pl.ds(jv * 64 + u * 16, 16)
                        eidx[sl4] = jnp.take(d, sub + 4 * u) * rw + lane
                        egidx[sl4] = jnp.take(ik, sub + 4 * u) * rw + lane
                return cc
            lax.fori_loop(0, nvreg, body, 0)

        def row_gather_desc(st):
            if rw == 16:
                return pltpu.make_async_copy(tab.at[st["gidx"]], st["rows"],
                                             st["semR"])
            return pltpu.make_async_copy(tab.at[st["egidx"]], st["rows"],
                                         st["semR"])

        def fire_gathers(st):
            row_gather_desc(st).start()
            pltpu.async_copy(qkt.at[st["ckb"]], st["qkv"], st["semQ"])

        def scat_desc(st):
            if rw == 16:
                return pltpu.make_async_copy(
                    st["rows"], agg_s.at[st["sidx"]], st["semS"])
            return pltpu.make_async_copy(
                st["rows"], agg_s.at[st["eidx"]], st["semS"])

        def wait_scatter(st):
            scat_desc(st).wait()
            pltpu.make_async_copy(st["ab"], den_s.at[st["sidx"]],
                                  st["semS"]).wait()

        def compute_and_scatter(st):
            qkv, ab, rows = st["qkv"], st["ab"], st["rows"]
            pltpu.make_async_copy(qkt.at[st["ckb"]], qkv, st["semQ"]).wait()

            def av(jv, cc):
                sl = pl.ds(jv * 16, 16)
                l = qkv[sl] + qkv[pl.ds(ch + jv * 16, 16)]
                l = jnp.where(l >= 0.0, l, 0.2 * l)
                ab[sl] = jnp.exp(l)
                return cc
            lax.fori_loop(0, nvreg, av, 0)
            row_gather_desc(st).wait()

            if rw == 16:
                def scale(jv, cc):
                    a16 = ab[pl.ds(jv * 16, 16)]
                    for l in range(16):
                        rows[jv * 16 + l, :] = rows[jv * 16 + l, :] * a16[l]
                    return cc
            else:
                def scale(jv, cc):
                    a16 = ab[pl.ds(jv * 16, 16)]
                    iotav = lax.iota(jnp.int32, 16)
                    sub = iotav >> 2
                    for u in range(4):
                        sl4 = pl.ds(jv * 64 + u * 16, 16)
                        rows[sl4] = rows[sl4] * jnp.take(a16, sub + 4 * u)
                    return cc
            lax.fori_loop(0, nvreg, scale, 0)

            scat_desc(st).start(add=True)
            pltpu.async_copy(ab, den_s.at[st["sidx"]], st["semS"], add=True)

        if rw == 16:
            for j in range(rpt // ZROWS):
                pltpu.sync_copy(z2d,
                                agg_s.at[pl.ds(rbase + j * ZROWS, ZROWS)])
        else:
            pltpu.sync_copy(z2d, agg_s.at[pl.ds(rbase * rw, rpt * rw)])
        pltpu.sync_copy(zfl, den_s.at[pl.ds(rbase, rpt)])
        plsc.subcore_barrier()

        fire_idx(0, A)
        wait_idx(A)
        vix(A)
        fire_gathers(A)
        fire_idx(1, B)

        def body(big, carry):
            i0 = 2 * big
            wait_idx(B)

            @pl.when(big > 0)
            def _():
                wait_scatter(B)
            vix(B)
            fire_gathers(B)

            @pl.when(i0 + 2 < nchunk)
            def _():
                fire_idx(i0 + 2, A)
            compute_and_scatter(A)
            @pl.when(i0 + 2 < nchunk)
            def _():
                wait_idx(A)
                wait_scatter(A)
                vix(A)
                fire_gathers(A)
                fire_idx(i0 + 3, B)
            compute_and_scatter(B)
            return carry
        lax.fori_loop(0, nchunk // 2, body, 0)
        wait_scatter(A)
        wait_scatter(B)
        plsc.subcore_barrier()

        if rw == 16:
            sl = pl.ds(rbase, rpt)
            pltpu.sync_copy(agg_s.at[sl], agg_out.at[c, sl])
        else:
            pltpu.sync_copy(
                agg_s.at[pl.ds(rbase * rw, rpt * rw)],
                agg_out.at[pl.ds((c * acc_rows + rbase) * rw, rpt * rw)])
        dbase = c * acc_rows + rbase
        pltpu.sync_copy(den_s.at[pl.ds(rbase, rpt)],
                        den_out.at[pl.ds(dbase, rpt)])

    return kern


_edge_pass_l1 = _edge_pass(L1_ROWS, gather_by_ik=False, ch=400, rw=16)
_edge_pass_l2 = _edge_pass(L2_ROWS, gather_by_ik=True, ch=1600, rw=4)

_B = 2000
_GRID = N // _B


def _full(shape):
    return pl.BlockSpec(shape, lambda i: (0,) * len(shape))


def _pre_kernel(x_ref, w1_ref, q1_ref, k1_ref, sq_ref, sk_ref):
    aq = jnp.concatenate([(w1_ref[0] @ q1_ref[0])[:, None],
                          (w1_ref[1] @ q1_ref[0])[:, None]], axis=1)
    ak = jnp.concatenate([(w1_ref[0] @ k1_ref[0])[:, None],
                          (w1_ref[1] @ k1_ref[0])[:, None]], axis=1)
    sq_ref[...] = x_ref[...] @ aq
    sk_ref[...] = x_ref[...] @ ak


def _mid_kernel(agg_ref, den_ref, w1_ref, b1_ref, w2_ref, q2_ref, k2_ref,
                xt2_ref, sq2_ref, sk2_ref):
    a = agg_ref[0] + agg_ref[1]
    h = a[:, 0, :] @ w1_ref[0] + a[:, 1, :] @ w1_ref[1]
    d = jnp.sum(den_ref[...], axis=(0, 2))
    h = h / (d[:, None] + 1e-16) + b1_ref[0][None, :]
    xt0 = h @ w2_ref[0]
    xt1 = h @ w2_ref[1]
    z = jnp.zeros((_B, 1, 4 - OUT), jnp.float32)
    xt2_ref[...] = jnp.concatenate(
        [xt0[:, None, :], z, xt1[:, None, :], z], axis=-1).reshape(_B, 2, 4)
    aq2 = jnp.concatenate([(w2_ref[0] @ q2_ref[0])[:, None],
                           (w2_ref[1] @ q2_ref[0])[:, None]], axis=1)
    ak2 = jnp.concatenate([(w2_ref[0] @ k2_ref[0])[:, None],
                           (w2_ref[1] @ k2_ref[0])[:, None]], axis=1)
    sq2_ref[...] = h @ aq2
    sk2_ref[...] = h @ ak2


def _post_kernel(acc_ref, den_ref, b2_ref, out_ref):
    a = acc_ref[0] + acc_ref[1]
    d = jnp.sum(den_ref[...], axis=(0, 2))
    out_ref[...] = a[:, :OUT] / (d[:, None] + 1e-16) + b2_ref[0][None, :]


def kernel(feature, edge_index, edge_type, W1, q1, k1, b1, W2, q2, k2, b2):
    src = edge_index[0]
    dst = edge_index[1]
    npad = E_PAD - E
    src_p = jnp.concatenate([src, jnp.zeros((npad,), jnp.int32)])
    dst_p = jnp.concatenate([dst, jnp.full((npad,), N, jnp.int32)])
    et_p = jnp.concatenate([edge_type, jnp.zeros((npad,), jnp.int32)])

    q1r = q1.reshape(1, HID)
    k1r = k1.reshape(1, HID)
    b1r = b1.reshape(1, HID)
    q2r = q2.reshape(1, OUT)
    k2r = k2.reshape(1, OUT)
    b2r = b2.reshape(1, OUT)

    sq1, sk1 = pl.pallas_call(
        _pre_kernel,
        grid=(_GRID,),
        in_specs=[
            pl.BlockSpec((_B, IN_DIM), lambda i: (i, 0)),
            _full((2, IN_DIM, HID)),
            _full((1, HID)),
            _full((1, HID)),
        ],
        out_specs=[
            pl.BlockSpec((_B, 2), lambda i: (i, 0)),
            pl.BlockSpec((_B, 2), lambda i: (i, 0)),
        ],
        out_shape=[
            jax.ShapeDtypeStruct((N, 2), jnp.float32),
            jax.ShapeDtypeStruct((N, 2), jnp.float32),
        ],
    )(feature, W1, q1r, k1r)

    e3 = jnp.stack([src_p, dst_p, et_p])
    qk1 = jnp.concatenate([sq1.reshape(2 * N), sk1.reshape(2 * N)])
    z16 = jnp.zeros((ZROWS, 16), jnp.float32)
    z4 = jnp.zeros((L2_ROWS // NS * 4,), jnp.float32)
    zf1 = jnp.zeros((L1_ROWS // NS,), jnp.float32)
    zf2 = jnp.zeros((L2_ROWS // NS,), jnp.float32)
    agg1, den1 = _edge_pass_l1(qk1, feature, e3, z16, zf1)

    agg1n = agg1[:, :2 * N].reshape(2, N, 2, 16)
    den1n = den1.reshape(2, L1_ROWS)[:, :2 * N].reshape(2, N, 2)

    xt2p, sq2, sk2 = pl.pallas_call(
        _mid_kernel,
        grid=(_GRID,),
        in_specs=[
            pl.BlockSpec((2, _B, 2, 16), lambda i: (0, i, 0, 0)),
            pl.BlockSpec((2, _B, 2), lambda i: (0, i, 0)),
            _full((2, IN_DIM, HID)),
            _full((1, HID)),
            _full((2, HID, OUT)),
            _full((1, OUT)),
            _full((1, OUT)),
        ],
        out_specs=[
            pl.BlockSpec((_B, 2, 4), lambda i: (i, 0, 0)),
            pl.BlockSpec((_B, 2), lambda i: (i, 0)),
            pl.BlockSpec((_B, 2), lambda i: (i, 0)),
        ],
        out_shape=[
            jax.ShapeDtypeStruct((N, 2, 4), jnp.float32),
            jax.ShapeDtypeStruct((N, 2), jnp.float32),
            jax.ShapeDtypeStruct((N, 2), jnp.float32),
        ],
    )(agg1n, den1n, W1, b1r, W2, q2r, k2r)

    qk2 = jnp.concatenate([sq2.reshape(2 * N), sk2.reshape(2 * N)])
    acc2, den2 = _edge_pass_l2(qk2, xt2p.reshape(2 * N * 4), e3, z4, zf2)

    acc2n = acc2.reshape(2, L2_ROWS, 4)[:, :N]
    den2n = den2.reshape(2, L2_ROWS)[:, :N].reshape(2, N, 1)

    out = pl.pallas_call(
        _post_kernel,
        grid=(_GRID,),
        in_specs=[
            pl.BlockSpec((2, _B, 4), lambda i: (0, i, 0)),
            pl.BlockSpec((2, _B, 1), lambda i: (0, i, 0)),
            _full((1, OUT)),
        ],
        out_specs=pl.BlockSpec((_B, OUT), lambda i: (i, 0)),
        out_shape=jax.ShapeDtypeStruct((N, OUT), jnp.float32),
    )(acc2n, den2n, b2r)

    return out


# speedup vs baseline: 1.2661x; 1.2661x over previous
import functools

import jax
import jax.numpy as jnp
from jax import lax
from jax.experimental import pallas as pl
from jax.experimental.pallas import tpu as pltpu
from jax.experimental.pallas import tpu_sc as plsc

N = 50000
E = 800000
IN_DIM = 16
HID = 64
OUT = 3

NC = 2
NS = 16
NW = NC * NS
EPT = 25600
E_PAD = NW * EPT
L1_ROWS = 2 * N + 96
L2_ROWS = N + 48
ZROWS = 1564


def _edge_pass(acc_rows, gather_by_ik, ch, rw):
    rpt = acc_rows // NS
    nchunk = EPT // ch
    nvreg = ch // 16
    mesh = plsc.VectorSubcoreMesh(core_axis_name="c", subcore_axis_name="s")

    nbuf = 2
    buf_types = []
    for _ in range(nbuf):
        buf_types += [
            pltpu.VMEM((3, ch), jnp.int32),
            pltpu.VMEM((2 * ch,), jnp.int32),
            pltpu.VMEM((ch,), jnp.int32),
            pltpu.VMEM((ch,), jnp.int32),
            pltpu.VMEM((2 * ch,), jnp.float32),
            pltpu.VMEM((ch,), jnp.float32),
            pltpu.VMEM((ch, 16), jnp.float32)
            if rw == 16 else pltpu.VMEM((ch * rw,), jnp.float32),
            pltpu.VMEM((ch * rw,), jnp.int32)
            if rw != 16 else pltpu.VMEM((8,), jnp.int32),
            pltpu.VMEM((ch * rw,), jnp.int32)
            if rw != 16 else pltpu.VMEM((8,), jnp.int32),
            pltpu.SemaphoreType.DMA,
            pltpu.SemaphoreType.DMA,
            pltpu.SemaphoreType.DMA,
            pltpu.SemaphoreType.DMA,
        ]

    @functools.partial(
        pl.kernel,
        mesh=mesh,
        compiler_params=pltpu.CompilerParams(use_tc_tiling_on_sc=False),
        out_type=[
            jax.ShapeDtypeStruct((NC, acc_rows, 16), jnp.float32)
            if rw == 16 else
            jax.ShapeDtypeStruct((NC * acc_rows * rw,), jnp.float32),
            jax.ShapeDtypeStruct((NC * acc_rows,), jnp.float32),
        ],
        scratch_types=[
            pltpu.VMEM_SHARED((acc_rows, 16), jnp.float32)
            if rw == 16 else
            pltpu.VMEM_SHARED((acc_rows * rw,), jnp.float32),
            pltpu.VMEM_SHARED((acc_rows,), jnp.float32),
        ] + buf_types,
    )
    def kern(qkt, tab, e3, z2d, zfl, agg_out, den_out, agg_s, den_s, *bufs):
        names = ("ebuf", "ckb", "gidx", "sidx", "qkv", "ab", "rows",
                 "egidx", "eidx", "semE", "semQ", "semR", "semS")
        A = dict(zip(names, bufs[:len(names)]))
        B = dict(zip(names, bufs[len(names):]))
        c = lax.axis_index("c")
        s = lax.axis_index("s")
        wid = c * NS + s
        tbase = wid * EPT
        rbase = s * rpt
        gmax = 2 * N - 1

        def fire_idx(i, st):
            pltpu.async_copy(e3.at[:, pl.ds(tbase + i * ch, ch)],
                             st["ebuf"], st["semE"])

        def wait_idx(st):
            pltpu.make_async_copy(e3.at[:, pl.ds(tbase, ch)],
                                  st["ebuf"], st["semE"]).wait()

        def vix(st):
            ebuf, ckb, gidx, sidx = st["ebuf"], st["ckb"], st["gidx"], st["sidx"]
            def body(jv, cc):
                sl = pl.ds(jv * 16, 16)
                sr = ebuf[0, sl]
                d = ebuf[1, sl]
                t = ebuf[2, sl]
                iq = 2 * d + t
                ik = 2 * sr + t
                ckb[sl] = jnp.minimum(iq, gmax)
                ckb[pl.ds(ch + jv * 16, 16)] = 2 * N + ik
                if gather_by_ik:
                    gidx[sl] = ik
                    sidx[sl] = d
                else:
                    gidx[sl] = sr
                    sidx[sl] = iq
                if rw != 16:
                    eidx = st["eidx"]
                    egidx = st["egidx"]
                    iotav = lax.iota(jnp.int32, 16)
                    sub = iotav >> 2
                    lane = iotav & 3
                    for u in range(4):
                        sl4 = pl.ds(jv * 64 + u * 16, 16)
                        eidx[sl4] = jnp.take(d, sub + 4 * u) * rw + lane
                        egidx[sl4] = jnp.take(ik, sub + 4 * u) * rw + lane
                return cc
            lax.fori_loop(0, nvreg, body, 0)

        def row_gather_desc(st):
            if rw == 16:
                return pltpu.make_async_copy(tab.at[st["gidx"]], st["rows"],
                                             st["semR"])
            return pltpu.make_async_copy(tab.at[st["egidx"]], st["rows"],
                                         st["semR"])

        def fire_gathers(st):
            row_gather_desc(st).start()
            pltpu.async_copy(qkt.at[st["ckb"]], st["qkv"], st["semQ"])

        def scat_desc(st):
            if rw == 16:
                return pltpu.make_async_copy(
                    st["rows"], agg_s.at[st["sidx"]], st["semS"])
            return pltpu.make_async_copy(
                st["rows"], agg_s.at[st["eidx"]], st["semS"])

        def wait_scatter(st):
            scat_desc(st).wait()
            pltpu.make_async_copy(st["ab"], den_s.at[st["sidx"]],
                                  st["semS"]).wait()

        def compute_and_scatter(st):
            qkv, ab, rows = st["qkv"], st["ab"], st["rows"]
            pltpu.make_async_copy(qkt.at[st["ckb"]], qkv, st["semQ"]).wait()

            def av(jv, cc):
                sl = pl.ds(jv * 16, 16)
                l = qkv[sl] + qkv[pl.ds(ch + jv * 16, 16)]
                l = jnp.where(l >= 0.0, l, 0.2 * l)
                ab[sl] = jnp.exp(l)
                return cc
            lax.fori_loop(0, nvreg, av, 0)
            row_gather_desc(st).wait()

            if rw == 16:
                def scale(jv, cc):
                    a16 = ab[pl.ds(jv * 16, 16)]
                    for l in range(16):
                        rows[jv * 16 + l, :] = rows[jv * 16 + l, :] * a16[l]
                    return cc
            else:
                def scale(jv, cc):
                    a16 = ab[pl.ds(jv * 16, 16)]
                    iotav = lax.iota(jnp.int32, 16)
                    sub = iotav >> 2
                    for u in range(4):
                        sl4 = pl.ds(jv * 64 + u * 16, 16)
                        rows[sl4] = rows[sl4] * jnp.take(a16, sub + 4 * u)
                    return cc
            lax.fori_loop(0, nvreg, scale, 0)

            scat_desc(st).start(add=True)
            pltpu.async_copy(ab, den_s.at[st["sidx"]], st["semS"], add=True)

        if rw == 16:
            for j in range(rpt // ZROWS):
                pltpu.sync_copy(z2d,
                                agg_s.at[pl.ds(rbase + j * ZROWS, ZROWS)])
        else:
            pltpu.sync_copy(z2d, agg_s.at[pl.ds(rbase * rw, rpt * rw)])
        pltpu.sync_copy(zfl, den_s.at[pl.ds(rbase, rpt)])
        plsc.subcore_barrier()

        fire_idx(0, A)
        wait_idx(A)
        vix(A)
        fire_gathers(A)
        fire_idx(1, B)

        def body(big, carry):
            i0 = 2 * big
            wait_idx(B)

            @pl.when(big > 0)
            def _():
                wait_scatter(B)
            vix(B)
            fire_gathers(B)

            @pl.when(i0 + 2 < nchunk)
            def _():
                fire_idx(i0 + 2, A)
            compute_and_scatter(A)
            @pl.when(i0 + 2 < nchunk)
            def _():
                wait_idx(A)
                wait_scatter(A)
                vix(A)
                fire_gathers(A)
                fire_idx(i0 + 3, B)
            compute_and_scatter(B)
            return carry
        lax.fori_loop(0, nchunk // 2, body, 0)
        wait_scatter(A)
        wait_scatter(B)
        plsc.subcore_barrier()

        if rw == 16:
            sl = pl.ds(rbase, rpt)
            pltpu.sync_copy(agg_s.at[sl], agg_out.at[c, sl])
        else:
            pltpu.sync_copy(
                agg_s.at[pl.ds(rbase * rw, rpt * rw)],
                agg_out.at[pl.ds((c * acc_rows + rbase) * rw, rpt * rw)])
        dbase = c * acc_rows + rbase
        pltpu.sync_copy(den_s.at[pl.ds(rbase, rpt)],
                        den_out.at[pl.ds(dbase, rpt)])

    return kern


_edge_pass_l1 = _edge_pass(L1_ROWS, gather_by_ik=False, ch=400, rw=16)
_edge_pass_l2 = _edge_pass(L2_ROWS, gather_by_ik=True, ch=1280, rw=16)

_B = 2000
_GRID = N // _B


def _full(shape):
    return pl.BlockSpec(shape, lambda i: (0,) * len(shape))


def _pre_kernel(x_ref, w1_ref, q1_ref, k1_ref, sq_ref, sk_ref):
    aq = jnp.concatenate([(w1_ref[0] @ q1_ref[0])[:, None],
                          (w1_ref[1] @ q1_ref[0])[:, None]], axis=1)
    ak = jnp.concatenate([(w1_ref[0] @ k1_ref[0])[:, None],
                          (w1_ref[1] @ k1_ref[0])[:, None]], axis=1)
    sq_ref[...] = x_ref[...] @ aq
    sk_ref[...] = x_ref[...] @ ak


def _mid_kernel(agg_ref, den_ref, w1_ref, b1_ref, w2_ref, q2_ref, k2_ref,
                xt2_ref, sq2_ref, sk2_ref):
    a = agg_ref[0] + agg_ref[1]
    h = a[:, 0, :] @ w1_ref[0] + a[:, 1, :] @ w1_ref[1]
    d = jnp.sum(den_ref[...], axis=(0, 2))
    h = h / (d[:, None] + 1e-16) + b1_ref[0][None, :]
    xt0 = h @ w2_ref[0]
    xt1 = h @ w2_ref[1]
    z = jnp.zeros((_B, 1, 16 - OUT), jnp.float32)
    xt2_ref[...] = jnp.concatenate(
        [xt0[:, None, :], z, xt1[:, None, :], z], axis=-1).reshape(_B, 2, 16)
    aq2 = jnp.concatenate([(w2_ref[0] @ q2_ref[0])[:, None],
                           (w2_ref[1] @ q2_ref[0])[:, None]], axis=1)
    ak2 = jnp.concatenate([(w2_ref[0] @ k2_ref[0])[:, None],
                           (w2_ref[1] @ k2_ref[0])[:, None]], axis=1)
    sq2_ref[...] = h @ aq2
    sk2_ref[...] = h @ ak2


def _post_kernel(acc_ref, den_ref, b2_ref, out_ref):
    a = acc_ref[0] + acc_ref[1]
    d = jnp.sum(den_ref[...], axis=(0, 2))
    out_ref[...] = a[:, :OUT] / (d[:, None] + 1e-16) + b2_ref[0][None, :]


def kernel(feature, edge_index, edge_type, W1, q1, k1, b1, W2, q2, k2, b2):
    src = edge_index[0]
    dst = edge_index[1]
    npad = E_PAD - E
    src_p = jnp.concatenate([src, jnp.zeros((npad,), jnp.int32)])
    dst_p = jnp.concatenate([dst, jnp.full((npad,), N, jnp.int32)])
    et_p = jnp.concatenate([edge_type, jnp.zeros((npad,), jnp.int32)])

    q1r = q1.reshape(1, HID)
    k1r = k1.reshape(1, HID)
    b1r = b1.reshape(1, HID)
    q2r = q2.reshape(1, OUT)
    k2r = k2.reshape(1, OUT)
    b2r = b2.reshape(1, OUT)

    sq1, sk1 = pl.pallas_call(
        _pre_kernel,
        grid=(_GRID,),
        in_specs=[
            pl.BlockSpec((_B, IN_DIM), lambda i: (i, 0)),
            _full((2, IN_DIM, HID)),
            _full((1, HID)),
            _full((1, HID)),
        ],
        out_specs=[
            pl.BlockSpec((_B, 2), lambda i: (i, 0)),
            pl.BlockSpec((_B, 2), lambda i: (i, 0)),
        ],
        out_shape=[
            jax.ShapeDtypeStruct((N, 2), jnp.float32),
            jax.ShapeDtypeStruct((N, 2), jnp.float32),
        ],
    )(feature, W1, q1r, k1r)

    e3 = jnp.stack([src_p, dst_p, et_p])
    qk1 = jnp.concatenate([sq1.reshape(2 * N), sk1.reshape(2 * N)])
    z16 = jnp.zeros((ZROWS, 16), jnp.float32)
    zf1 = jnp.zeros((L1_ROWS // NS,), jnp.float32)
    zf2 = jnp.zeros((L2_ROWS // NS,), jnp.float32)
    agg1, den1 = _edge_pass_l1(qk1, feature, e3, z16, zf1)

    agg1n = agg1[:, :2 * N].reshape(2, N, 2, 16)
    den1n = den1.reshape(2, L1_ROWS)[:, :2 * N].reshape(2, N, 2)

    xt2p, sq2, sk2 = pl.pallas_call(
        _mid_kernel,
        grid=(_GRID,),
        in_specs=[
            pl.BlockSpec((2, _B, 2, 16), lambda i: (0, i, 0, 0)),
            pl.BlockSpec((2, _B, 2), lambda i: (0, i, 0)),
            _full((2, IN_DIM, HID)),
            _full((1, HID)),
            _full((2, HID, OUT)),
            _full((1, OUT)),
            _full((1, OUT)),
        ],
        out_specs=[
            pl.BlockSpec((_B, 2, 16), lambda i: (i, 0, 0)),
            pl.BlockSpec((_B, 2), lambda i: (i, 0)),
            pl.BlockSpec((_B, 2), lambda i: (i, 0)),
        ],
        out_shape=[
            jax.ShapeDtypeStruct((N, 2, 16), jnp.float32),
            jax.ShapeDtypeStruct((N, 2), jnp.float32),
            jax.ShapeDtypeStruct((N, 2), jnp.float32),
        ],
    )(agg1n, den1n, W1, b1r, W2, q2r, k2r)

    qk2 = jnp.concatenate([sq2.reshape(2 * N), sk2.reshape(2 * N)])
    acc2, den2 = _edge_pass_l2(qk2, xt2p.reshape(2 * N, 16), e3, z16, zf2)

    acc2n = acc2[:, :N]
    den2n = den2.reshape(2, L2_ROWS)[:, :N].reshape(2, N, 1)

    out = pl.pallas_call(
        _post_kernel,
        grid=(_GRID,),
        in_specs=[
            pl.BlockSpec((2, _B, 16), lambda i: (0, i, 0)),
            pl.BlockSpec((2, _B, 1), lambda i: (0, i, 0)),
            _full((1, OUT)),
        ],
        out_specs=pl.BlockSpec((_B, OUT), lambda i: (i, 0)),
        out_shape=jax.ShapeDtypeStruct((N, OUT), jnp.float32),
    )(acc2n, den2n, b2r)

    return out
